# baseline (speedup 1.0000x reference)
import jax
import jax.numpy as jnp
from jax import lax
from jax.experimental import pallas as pl
from jax.experimental.pallas import tpu as pltpu

B, C, H, W = 128, 768, 14, 14
HW = H * W
E, TOPK = 16, 2

KB = 7
NSTEP = HW // KB // 4


def _gate_body(xa_ref, xb_ref, xc_ref, xd_ref, w1_ref, b1_ref, w2_ref,
               b2_ref, noise_ref, w_out, idx_out, s_ref, m_ref):
    k = pl.program_id(0)
    xa = xa_ref[...]
    xb = xb_ref[...]
    xc = xc_ref[...]
    xd = xd_ref[...]
    ps = ((jnp.sum(xa, axis=0) + jnp.sum(xb, axis=0))
          + (jnp.sum(xc, axis=0) + jnp.sum(xd, axis=0)))
    pm = jnp.maximum(jnp.maximum(jnp.max(xa, axis=0), jnp.max(xb, axis=0)),
                     jnp.maximum(jnp.max(xc, axis=0), jnp.max(xd, axis=0)))

    @pl.when(k == 0)
    def _init():
        s_ref[...] = ps
        m_ref[...] = pm

    @pl.when(k > 0)
    def _acc():
        s_ref[...] += ps
        m_ref[...] = jnp.maximum(m_ref[...], pm)

    @pl.when(k == NSTEP - 1)
    def _finish():
        f = s_ref[...] * (1.0 / HW) + m_ref[...]
        fb = f.astype(jnp.bfloat16)
        dn = (((1,), (1,)), ((), ()))
        z1 = lax.dot_general(
            fb, w1_ref[...].astype(jnp.bfloat16), dimension_numbers=dn,
            preferred_element_type=jnp.float32,
        ) + b1_ref[...]
        z2 = lax.dot_general(
            fb, w2_ref[...].astype(jnp.bfloat16), dimension_numbers=dn,
            preferred_element_type=jnp.float32,
        ) + b2_ref[...]

        n1 = z1
        n2 = z2
        nz = noise_ref[...].T
        n = n1 + nz * jax.nn.softplus(n2)

        iota = lax.broadcasted_iota(jnp.int32, (B, E), 1)
        v1 = jnp.max(n, axis=1, keepdims=True)
        i1 = jnp.min(jnp.where(n == v1, iota, E), axis=1, keepdims=True)
        masked = jnp.where(iota == i1, -jnp.inf, n)
        v2 = jnp.max(masked, axis=1, keepdims=True)
        i2 = jnp.min(jnp.where(masked == v2, iota, E), axis=1, keepdims=True)

        e2 = jnp.exp(v2 - v1)
        denom = 1.0 + e2
        w_out[...] = jnp.where(
            iota == i1, 1.0 / denom,
            jnp.where(iota == i2, e2 / denom, 0.0))
        idx_out[...] = jnp.concatenate([i1, i2], axis=1)


@jax.jit
def kernel(x, w1_w, w1_b, w2_w, w2_b, noise):
    xt = jnp.transpose(x, (2, 3, 0, 1)).reshape(HW, B, C)

    grid = (NSTEP,)
    w, idx = pl.pallas_call(
        _gate_body,
        grid=grid,
        in_specs=[
            pl.BlockSpec((KB, B, C), lambda k: (k, 0, 0)),
            pl.BlockSpec((KB, B, C), lambda k: (k + NSTEP, 0, 0)),
            pl.BlockSpec((KB, B, C), lambda k: (k + 2 * NSTEP, 0, 0)),
            pl.BlockSpec((KB, B, C), lambda k: (k + 3 * NSTEP, 0, 0)),
            pl.BlockSpec((E, C), lambda k: (0, 0)),
            pl.BlockSpec((1, E), lambda k: (0, 0)),
            pl.BlockSpec((E, C), lambda k: (0, 0)),
            pl.BlockSpec((1, E), lambda k: (0, 0)),
            pl.BlockSpec((E, B), lambda k: (0, 0)),
        ],
        out_specs=[
            pl.BlockSpec((B, E), lambda k: (0, 0)),
            pl.BlockSpec((B, TOPK), lambda k: (0, 0)),
        ],
        out_shape=[
            jax.ShapeDtypeStruct((B, E), jnp.float32),
            jax.ShapeDtypeStruct((B, TOPK), jnp.int32),
        ],
        scratch_shapes=[
            pltpu.VMEM((B, C), jnp.float32),
            pltpu.VMEM((B, C), jnp.float32),
        ],
    )(xt, xt, xt, xt, w1_w, w1_b.reshape(1, E), w2_w, w2_b.reshape(1, E), noise.T)
    return (w, idx)


# speedup vs baseline: 1.0138x; 1.0138x over previous
import jax
import jax.numpy as jnp
from jax import lax
from jax.experimental import pallas as pl
from jax.experimental.pallas import tpu as pltpu

B, C, H, W = 128, 768, 14, 14
HW = H * W
E, TOPK = 16, 2

KB = 14
NSTEP = HW // KB // 2


def _gate_body(xa_ref, xb_ref, w1_ref, b1_ref, w2_ref, b2_ref, noise_ref,
               w_out, idx_out, s_ref, m_ref):
    k = pl.program_id(0)
    xa = xa_ref[...]
    xb = xb_ref[...]
    ps = jnp.sum(xa, axis=0) + jnp.sum(xb, axis=0)
    pm = jnp.maximum(jnp.max(xa, axis=0), jnp.max(xb, axis=0))

    @pl.when(k == 0)
    def _init():
        s_ref[...] = ps
        m_ref[...] = pm

    @pl.when(k > 0)
    def _acc():
        s_ref[...] += ps
        m_ref[...] = jnp.maximum(m_ref[...], pm)

    @pl.when(k == NSTEP - 1)
    def _finish():
        f = s_ref[...] * (1.0 / HW) + m_ref[...]
        fb = f.astype(jnp.bfloat16)
        dn = (((1,), (1,)), ((), ()))
        z1 = lax.dot_general(
            fb, w1_ref[...].astype(jnp.bfloat16), dimension_numbers=dn,
            preferred_element_type=jnp.float32,
        ) + b1_ref[...]
        z2 = lax.dot_general(
            fb, w2_ref[...].astype(jnp.bfloat16), dimension_numbers=dn,
            preferred_element_type=jnp.float32,
        ) + b2_ref[...]

        n1 = z1
        n2 = z2
        nz = noise_ref[...].T
        n = n1 + nz * jax.nn.softplus(n2)

        iota = lax.broadcasted_iota(jnp.int32, (B, E), 1)
        v1 = jnp.max(n, axis=1, keepdims=True)
        i1 = jnp.min(jnp.where(n == v1, iota, E), axis=1, keepdims=True)
        masked = jnp.where(iota == i1, -jnp.inf, n)
        v2 = jnp.max(masked, axis=1, keepdims=True)
        i2 = jnp.min(jnp.where(masked == v2, iota, E), axis=1, keepdims=True)

        e2 = jnp.exp(v2 - v1)
        denom = 1.0 + e2
        w_out[...] = jnp.where(
            iota == i1, 1.0 / denom,
            jnp.where(iota == i2, e2 / denom, 0.0))
        idx_out[...] = jnp.concatenate([i1, i2], axis=1)


@jax.jit
def kernel(x, w1_w, w1_b, w2_w, w2_b, noise):
    xt = jnp.transpose(x, (2, 3, 0, 1)).reshape(HW, B, C)

    grid = (NSTEP,)
    w, idx = pl.pallas_call(
        _gate_body,
        grid=grid,
        in_specs=[
            pl.BlockSpec((KB, B, C), lambda k: (k, 0, 0)),
            pl.BlockSpec((KB, B, C), lambda k: (k + NSTEP, 0, 0)),
            pl.BlockSpec((E, C), lambda k: (0, 0)),
            pl.BlockSpec((1, E), lambda k: (0, 0)),
            pl.BlockSpec((E, C), lambda k: (0, 0)),
            pl.BlockSpec((1, E), lambda k: (0, 0)),
            pl.BlockSpec((E, B), lambda k: (0, 0)),
        ],
        out_specs=[
            pl.BlockSpec((B, E), lambda k: (0, 0)),
            pl.BlockSpec((B, TOPK), lambda k: (0, 0)),
        ],
        out_shape=[
            jax.ShapeDtypeStruct((B, E), jnp.float32),
            jax.ShapeDtypeStruct((B, TOPK), jnp.int32),
        ],
        scratch_shapes=[
            pltpu.VMEM((B, C), jnp.float32),
            pltpu.VMEM((B, C), jnp.float32),
        ],
    )(xt, xt, w1_w, w1_b.reshape(1, E), w2_w, w2_b.reshape(1, E), noise.T)
    return (w, idx)


# speedup vs baseline: 1.0475x; 1.0332x over previous
import jax
import jax.numpy as jnp
from jax import lax
from jax.experimental import pallas as pl
from jax.experimental.pallas import tpu as pltpu

B, C, H, W = 128, 768, 14, 14
HW = H * W
E, TOPK = 16, 2

KB = 14
NSTEP = HW // KB // 2


def _gate_body(xa_ref, xb_ref, w1_ref, b1_ref, w2_ref, b2_ref, noise_ref,
               w_out, idx_out, s_ref, m_ref):
    k = pl.program_id(0)
    xa = xa_ref[...]
    xb = xb_ref[...]
    ps = jnp.sum(xa, axis=0) + jnp.sum(xb, axis=0)
    pm = jnp.maximum(jnp.max(xa, axis=0), jnp.max(xb, axis=0))

    @pl.when(k == 0)
    def _init():
        s_ref[...] = ps
        m_ref[...] = pm

    @pl.when(k > 0)
    def _acc():
        s_ref[...] += ps
        m_ref[...] = jnp.maximum(m_ref[...], pm)

    @pl.when(k == NSTEP - 1)
    def _finish():
        f = s_ref[...] * (1.0 / HW) + m_ref[...]
        fb = f.astype(jnp.bfloat16)
        dn = (((1,), (1,)), ((), ()))
        z1 = lax.dot_general(
            w1_ref[...].astype(jnp.bfloat16), fb, dimension_numbers=dn,
            preferred_element_type=jnp.float32,
        ) + b1_ref[...]
        z2 = lax.dot_general(
            w2_ref[...].astype(jnp.bfloat16), fb, dimension_numbers=dn,
            preferred_element_type=jnp.float32,
        ) + b2_ref[...]

        n = z1 + noise_ref[...] * jax.nn.softplus(z2)

        iota = lax.broadcasted_iota(jnp.int32, (E, B), 0)
        v1 = jnp.max(n, axis=0, keepdims=True)
        i1 = jnp.min(jnp.where(n == v1, iota, E), axis=0, keepdims=True)
        masked = jnp.where(iota == i1, -jnp.inf, n)
        v2 = jnp.max(masked, axis=0, keepdims=True)
        i2 = jnp.min(jnp.where(masked == v2, iota, E), axis=0, keepdims=True)

        e2 = jnp.exp(v2 - v1)
        denom = 1.0 + e2
        w_out[...] = jnp.where(
            iota == i1, 1.0 / denom,
            jnp.where(iota == i2, e2 / denom, 0.0))
        idx_out[...] = jnp.concatenate([i1, i2], axis=0)


@jax.jit
def kernel(x, w1_w, w1_b, w2_w, w2_b, noise):
    xt = jnp.transpose(x, (2, 3, 0, 1)).reshape(HW, B, C)

    grid = (NSTEP,)
    w, idx = pl.pallas_call(
        _gate_body,
        grid=grid,
        in_specs=[
            pl.BlockSpec((KB, B, C), lambda k: (k, 0, 0)),
            pl.BlockSpec((KB, B, C), lambda k: (k + NSTEP, 0, 0)),
            pl.BlockSpec((E, C), lambda k: (0, 0)),
            pl.BlockSpec((E, 1), lambda k: (0, 0)),
            pl.BlockSpec((E, C), lambda k: (0, 0)),
            pl.BlockSpec((E, 1), lambda k: (0, 0)),
            pl.BlockSpec((E, B), lambda k: (0, 0)),
        ],
        out_specs=[
            pl.BlockSpec((E, B), lambda k: (0, 0)),
            pl.BlockSpec((TOPK, B), lambda k: (0, 0)),
        ],
        out_shape=[
            jax.ShapeDtypeStruct((E, B), jnp.float32),
            jax.ShapeDtypeStruct((TOPK, B), jnp.int32),
        ],
        scratch_shapes=[
            pltpu.VMEM((B, C), jnp.float32),
            pltpu.VMEM((B, C), jnp.float32),
        ],
    )(xt, xt, w1_w, w1_b.reshape(E, 1), w2_w, w2_b.reshape(E, 1), noise.T)
    return (w.T, idx.T)
